# Initial kernel scaffold; baseline (speedup 1.0000x reference)
import jax
import jax.numpy as jnp
from jax.experimental import pallas as pl


def kernel(x, edge_index, edge_attr, pos, batch, W1, b1, nW1, nb1, nW2, nb2, Wr, br, Wc, W_ih, W_hh, b_lstm, Wo1, bo1, Wo2, bo2):
    raise NotImplementedError("write your pallas kernel here")


# speedup vs baseline: 2.6135x; 2.6135x over previous
import functools

import jax
import jax.numpy as jnp
from jax import lax
from jax.experimental import pallas as pl
from jax.experimental.pallas import tpu as pltpu
from jax.experimental.pallas import tpu_sc as plsc

_NC = 2
_NS = 16
_NW = _NC * _NS
_CH = 128

_INTERPRET = False


def _cdiv(a, b):
    return (a + b - 1) // b


def _sc_gather(table, posd, srcp, dstp):
    epad = srcp.shape[0]
    per_w = epad // _NW
    nchunks = per_w // _CH
    mesh = plsc.VectorSubcoreMesh(core_axis_name="c", subcore_axis_name="s")

    def body(table_h, posd_h, src_h, dst_h, gsrc_h, gdst_h,
             idx_s, idx_d, bufs, bufd, sem_s, sem_d):
        wid = lax.axis_index("s") * _NC + lax.axis_index("c")
        base = wid * per_w

        def step(i, carry):
            off = base + i * _CH
            pltpu.sync_copy(src_h.at[pl.ds(off, _CH)], idx_s)
            pltpu.async_copy(table_h.at[idx_s], bufs, sem_s).wait()
            pltpu.sync_copy(bufs, gsrc_h.at[pl.ds(off, _CH)])
            pltpu.sync_copy(dst_h.at[pl.ds(off, _CH)], idx_d)
            pltpu.async_copy(posd_h.at[idx_d], bufd, sem_d).wait()
            pltpu.sync_copy(bufd, gdst_h.at[pl.ds(off, _CH)])
            return carry

        lax.fori_loop(0, nchunks, step, 0)

    f = pl.kernel(
        body,
        out_type=(jax.ShapeDtypeStruct((epad, 32), jnp.float32),
                  jax.ShapeDtypeStruct((epad, 16), jnp.float32)),
        mesh=mesh,
        scratch_types=[
            pltpu.VMEM((_CH,), jnp.int32),
            pltpu.VMEM((_CH,), jnp.int32),
            pltpu.VMEM((_CH, 32), jnp.float32),
            pltpu.VMEM((_CH, 16), jnp.float32),
            pltpu.SemaphoreType.DMA,
            pltpu.SemaphoreType.DMA,
        ],
        compiler_params=pltpu.CompilerParams(use_tc_tiling_on_sc=False),
        interpret=_INTERPRET,
    )
    return f(table, posd, srcp, dstp)


def _sc_scatter(payload, dstp, zinit):
    epad = payload.shape[0]
    nacc = zinit.shape[0]
    per_w = epad // _NW
    nchunks = per_w // _CH
    rpt = nacc // _NS
    mesh = plsc.VectorSubcoreMesh(core_axis_name="c", subcore_axis_name="s")

    def body(pay_h, dst_h, z_h, out_h, idx_v, pay_v, sem, accum):
        cid = lax.axis_index("c")
        sid = lax.axis_index("s")
        wid = sid * _NC + cid
        pltpu.sync_copy(z_h.at[pl.ds(sid * rpt, rpt)],
                        accum.at[pl.ds(sid * rpt, rpt)])
        plsc.subcore_barrier()

        def step(i, carry):
            off = wid * per_w + i * _CH
            pltpu.sync_copy(dst_h.at[pl.ds(off, _CH)], idx_v)
            pltpu.sync_copy(pay_h.at[pl.ds(off, _CH)], pay_v)
            pltpu.sync_copy(pay_v, accum.at[idx_v], add=True)
            return carry

        lax.fori_loop(0, nchunks, step, 0)
        plsc.subcore_barrier()
        pltpu.sync_copy(accum.at[pl.ds(sid * rpt, rpt)],
                        out_h.at[cid, pl.ds(sid * rpt, rpt)])

    f = pl.kernel(
        body,
        out_type=jax.ShapeDtypeStruct((_NC, nacc, 32), jnp.float32),
        mesh=mesh,
        scratch_types=[
            pltpu.VMEM((_CH,), jnp.int32),
            pltpu.VMEM((_CH, 32), jnp.float32),
            pltpu.SemaphoreType.DMA,
            pltpu.VMEM_SHARED((nacc, 32), jnp.float32),
        ],
        compiler_params=pltpu.CompilerParams(use_tc_tiling_on_sc=False),
        interpret=_INTERPRET,
    )
    return f(payload, dstp, zinit)


def _sigmoid(v):
    return 1.0 / (1.0 + jnp.exp(-v))


def _tc_init(x, pos, W1, b1r, tn):
    n = x.shape[0]
    din = x.shape[1]
    grid = (n // tn,)

    def body(x_r, p_r, w_r, b_r, tab_r, posd_r):
        h = jnp.dot(x_r[...], w_r[...]) + b_r[...]
        p = p_r[...]
        z13 = jnp.zeros((tn, 13), jnp.float32)
        tab_r[...] = jnp.concatenate([h, p, z13], axis=1)
        posd_r[...] = jnp.concatenate([p, z13], axis=1)

    return pl.pallas_call(
        body,
        grid=grid,
        in_specs=[
            pl.BlockSpec((tn, din), lambda i: (i, 0)),
            pl.BlockSpec((tn, 3), lambda i: (i, 0)),
            pl.BlockSpec(W1.shape, lambda i: (0, 0)),
            pl.BlockSpec(b1r.shape, lambda i: (0, 0)),
        ],
        out_specs=[
            pl.BlockSpec((tn, 32), lambda i: (i, 0)),
            pl.BlockSpec((tn, 16), lambda i: (i, 0)),
        ],
        out_shape=[
            jax.ShapeDtypeStruct((n, 32), jnp.float32),
            jax.ShapeDtypeStruct((n, 16), jnp.float32),
        ],
        interpret=_INTERPRET,
    )(x, pos, W1, b1r)


def _tc_edge(gsrc, gdst, eap, nW1a, nW1b, nb1r, nW2, nb2r, tile_m, sel_m,
             wc_l, teb):
    epad = gsrc.shape[0]
    dedge = eap.shape[1]
    grid = (epad // teb,)

    def body(gs_r, gd_r, ea_r, w1a_r, w1b_r, b1_r, w2_r, b2_r, tl_r, sl_r,
             wc_r, out_r):
        gs = gs_r[...]
        hs = gs[:, 0:16]
        rel = gs[:, 16:19] - gd_r[...][:, 0:3]
        dist = jnp.sqrt(jnp.sum(rel * rel, axis=1, keepdims=True) + 1e-12)
        zpre = (jnp.dot(ea_r[...], w1a_r[...]) + dist * w1b_r[...]
                + b1_r[...])
        z = zpre * _sigmoid(zpre)
        wef = jnp.dot(z, w2_r[...]) + b2_r[...]
        hst = jnp.dot(hs, tl_r[...])
        msg = jnp.dot(wef * hst, sl_r[...])
        cw = jnp.dot(msg, wc_r[...])
        wmsg = rel * cw
        ones = jnp.ones((teb, 1), jnp.float32)
        pad = jnp.zeros((teb, 12), jnp.float32)
        out_r[...] = jnp.concatenate([msg, wmsg, ones, pad], axis=1)

    return pl.pallas_call(
        body,
        grid=grid,
        in_specs=[
            pl.BlockSpec((teb, 32), lambda i: (i, 0)),
            pl.BlockSpec((teb, 16), lambda i: (i, 0)),
            pl.BlockSpec((teb, dedge), lambda i: (i, 0)),
            pl.BlockSpec(nW1a.shape, lambda i: (0, 0)),
            pl.BlockSpec(nW1b.shape, lambda i: (0, 0)),
            pl.BlockSpec(nb1r.shape, lambda i: (0, 0)),
            pl.BlockSpec(nW2.shape, lambda i: (0, 0)),
            pl.BlockSpec(nb2r.shape, lambda i: (0, 0)),
            pl.BlockSpec(tile_m.shape, lambda i: (0, 0)),
            pl.BlockSpec(sel_m.shape, lambda i: (0, 0)),
            pl.BlockSpec(wc_l.shape, lambda i: (0, 0)),
        ],
        out_specs=pl.BlockSpec((teb, 32), lambda i: (i, 0)),
        out_shape=jax.ShapeDtypeStruct((epad, 32), jnp.float32),
        compiler_params=pltpu.CompilerParams(
            dimension_semantics=("arbitrary",)),
        interpret=_INTERPRET,
    )(gsrc, gdst, eap, nW1a, nW1b, nb1r, nW2, nb2r, tile_m, sel_m, wc_l)


def _tc_update(table, agg2, Wr_l, br_l, tn):
    n = table.shape[0]
    nacc = agg2.shape[1]
    grid = (n // tn,)

    def body(tab_r, agg_r, wr_r, br_r, tabo_r, posdo_r):
        agg = agg_r[0] + agg_r[1]
        cnt = agg[:, 19:20]
        deg = jnp.maximum(cnt, 1.0)
        aggh = agg[:, 0:16] / deg
        aggp = agg[:, 16:19] / deg
        tab = tab_r[...]
        h = tab[:, 0:16]
        p = tab[:, 16:19]
        hn = h + jnp.dot(h, wr_r[...]) + aggh + br_r[...]
        pn = p + aggp
        z13 = jnp.zeros((tn, 13), jnp.float32)
        tabo_r[...] = jnp.concatenate([hn, pn, z13], axis=1)
        posdo_r[...] = jnp.concatenate([pn, z13], axis=1)

    return pl.pallas_call(
        body,
        grid=grid,
        in_specs=[
            pl.BlockSpec((tn, 32), lambda i: (i, 0)),
            pl.BlockSpec((2, tn, 32), lambda i: (0, i, 0)),
            pl.BlockSpec(Wr_l.shape, lambda i: (0, 0)),
            pl.BlockSpec(br_l.shape, lambda i: (0, 0)),
        ],
        out_specs=[
            pl.BlockSpec((tn, 32), lambda i: (i, 0)),
            pl.BlockSpec((tn, 16), lambda i: (i, 0)),
        ],
        out_shape=[
            jax.ShapeDtypeStruct((n, 32), jnp.float32),
            jax.ShapeDtypeStruct((n, 16), jnp.float32),
        ],
        interpret=_INTERPRET,
    )(table, agg2, Wr_l, br_l)


def _tc_set2set(table, batch2, W_ih, W_hh, b_lstm_r, Wo1, bo1r, Wo2, bo2r,
                bgraph, msteps):
    n = table.shape[0]
    hdim = 16

    def body(tab_r, bat_r, wih_r, whh_r, bl_r, wo1_r, bo1_r, wo2_r, bo2_r,
             out_r):
        h = tab_r[...][:, 0:hdim]
        bat = bat_r[...]
        ids = lax.broadcasted_iota(jnp.int32, (1, bgraph), 1)
        pm = bat == ids
        qstar = jnp.zeros((bgraph, 2 * hdim), jnp.float32)
        hs = jnp.zeros((bgraph, hdim), jnp.float32)
        cs = jnp.zeros((bgraph, hdim), jnp.float32)
        for _ in range(msteps):
            gates = (jnp.dot(qstar, wih_r[...]) + jnp.dot(hs, whh_r[...])
                     + bl_r[...])
            gi = gates[:, 0:hdim]
            gf = gates[:, hdim:2 * hdim]
            gg = gates[:, 2 * hdim:3 * hdim]
            go = gates[:, 3 * hdim:4 * hdim]
            cs = _sigmoid(gf) * cs + _sigmoid(gi) * jnp.tanh(gg)
            hs = _sigmoid(go) * jnp.tanh(cs)
            s = lax.dot_general(h, hs, (((1,), (1,)), ((), ())))
            masked = jnp.where(pm, s, -jnp.inf)
            emax = jnp.max(masked, axis=0, keepdims=True)
            emax = jnp.where(emax > -jnp.inf, emax, 0.0)
            a = jnp.exp(masked - emax)
            asum = jnp.sum(a, axis=0, keepdims=True)
            asum = jnp.where(asum > 0.0, asum, 1.0)
            an = a / asum
            r = lax.dot_general(an, h, (((0,), (0,)), ((), ())))
            qstar = jnp.concatenate([hs, r], axis=1)
        t = jnp.dot(qstar, wo1_r[...]) + bo1_r[...]
        sil = t * _sigmoid(t)
        out_r[...] = jnp.dot(sil, wo2_r[...]) + bo2_r[...]

    return pl.pallas_call(
        body,
        out_shape=jax.ShapeDtypeStruct((bgraph, 1), jnp.float32),
        interpret=_INTERPRET,
    )(table, batch2, W_ih, W_hh, b_lstm_r, Wo1, bo1r, Wo2, bo2r)


def kernel(x, edge_index, edge_attr, pos, batch, W1, b1, nW1, nb1, nW2, nb2,
           Wr, br, Wc, W_ih, W_hh, b_lstm, Wo1, bo1, Wo2, bo2):
    n, din = x.shape
    e = edge_index.shape[1]
    dedge = edge_attr.shape[1]
    hdim = W1.shape[1]
    nlayers = Wr.shape[0]
    bgraph = 64
    msteps = 3
    tn = 2000

    epad = _cdiv(e, _NW * _CH) * (_NW * _CH)
    nacc = n + 16

    src = edge_index[0].astype(jnp.int32)
    dst = edge_index[1].astype(jnp.int32)
    srcp = jnp.concatenate([src, jnp.zeros((epad - e,), jnp.int32)])
    dstp = jnp.concatenate([dst, jnp.full((epad - e,), n, jnp.int32)])
    eap = jnp.concatenate(
        [edge_attr, jnp.zeros((epad - e, dedge), jnp.float32)], axis=0)
    zinit = jnp.zeros((nacc, 32), jnp.float32)

    eye = jnp.eye(hdim, dtype=jnp.float32)
    tile_m = jnp.tile(eye, (1, hdim))
    sel_m = jnp.repeat(eye, hdim, axis=0)

    nW1a = nW1[:dedge]
    nW1b = nW1[dedge:dedge + 1]
    nb1r = nb1.reshape(1, -1)
    nb2r = nb2.reshape(1, -1)

    table, posd = _tc_init(x, pos, W1, b1.reshape(1, -1), tn)
    for l in range(nlayers):
        gsrc, gdst = _sc_gather(table, posd, srcp, dstp)
        payload = _tc_edge(gsrc, gdst, eap, nW1a, nW1b, nb1r, nW2, nb2r,
                           tile_m, sel_m, Wc[l], 2048)
        agg2 = _sc_scatter(payload, dstp, zinit)
        table, posd = _tc_update(table, agg2, Wr[l], br[l].reshape(1, -1), tn)

    out = _tc_set2set(table, batch.reshape(-1, 1).astype(jnp.int32),
                      W_ih, W_hh, b_lstm.reshape(1, -1), Wo1,
                      bo1.reshape(1, -1), Wo2, bo2.reshape(1, 1),
                      bgraph, msteps)
    return out.reshape(-1)


# speedup vs baseline: 2.9153x; 1.1155x over previous
import functools

import jax
import jax.numpy as jnp
from jax import lax
from jax.experimental import pallas as pl
from jax.experimental.pallas import tpu as pltpu
from jax.experimental.pallas import tpu_sc as plsc

_NC = 2
_NS = 16
_NW = _NC * _NS
_CH = 128

_INTERPRET = False


def _cdiv(a, b):
    return (a + b - 1) // b


_GDEPTH = 8


def _sc_gather(table, posd, srcp, dstp):
    epad = srcp.shape[0]
    per_w = epad // _NW
    nchunks = per_w // _CH
    ngroups = nchunks // _GDEPTH
    src2 = srcp.reshape(-1, _CH)
    dst2 = dstp.reshape(-1, _CH)
    mesh = plsc.VectorSubcoreMesh(core_axis_name="c", subcore_axis_name="s")

    def body(table_h, posd_h, src_h, dst_h, gsrc_h, gdst_h,
             idx_s, idx_d, bufs, bufd, gsem, wsem):
        wid = lax.axis_index("s") * _NC + lax.axis_index("c")
        base = wid * per_w
        crow = wid * nchunks
        pltpu.sync_copy(src_h.at[pl.ds(crow, nchunks)], idx_s)
        pltpu.sync_copy(dst_h.at[pl.ds(crow, nchunks)], idx_d)
        wdescs = []
        for g in range(ngroups):
            for d in wdescs:
                d.wait()
            wdescs = []
            gdescs = []
            for b in range(_GDEPTH):
                j = g * _GDEPTH + b
                gdescs.append(pltpu.async_copy(
                    table_h.at[idx_s.at[j]], bufs.at[b], gsem))
                gdescs.append(pltpu.async_copy(
                    posd_h.at[idx_d.at[j]], bufd.at[b], gsem))
            for d in gdescs:
                d.wait()
            for b in range(_GDEPTH):
                off = base + (g * _GDEPTH + b) * _CH
                wdescs.append(pltpu.async_copy(
                    bufs.at[b], gsrc_h.at[pl.ds(off, _CH)], wsem))
                wdescs.append(pltpu.async_copy(
                    bufd.at[b], gdst_h.at[pl.ds(off, _CH)], wsem))
        for d in wdescs:
            d.wait()

    f = pl.kernel(
        body,
        out_type=(jax.ShapeDtypeStruct((epad, 32), jnp.float32),
                  jax.ShapeDtypeStruct((epad, 16), jnp.float32)),
        mesh=mesh,
        scratch_types=[
            pltpu.VMEM((nchunks, _CH), jnp.int32),
            pltpu.VMEM((nchunks, _CH), jnp.int32),
            pltpu.VMEM((_GDEPTH, _CH, 32), jnp.float32),
            pltpu.VMEM((_GDEPTH, _CH, 16), jnp.float32),
            pltpu.SemaphoreType.DMA,
            pltpu.SemaphoreType.DMA,
        ],
        compiler_params=pltpu.CompilerParams(use_tc_tiling_on_sc=False),
        interpret=_INTERPRET,
    )
    return f(table, posd, src2, dst2)


def _sc_scatter(payload, dstp, zinit):
    epad = payload.shape[0]
    nacc = zinit.shape[0]
    per_w = epad // _NW
    nchunks = per_w // _CH
    rpt = nacc // _NS
    mesh = plsc.VectorSubcoreMesh(core_axis_name="c", subcore_axis_name="s")

    dst2 = dstp.reshape(-1, _CH)

    def body(pay_h, dst_h, z_h, out_h, idx_v, pay_v, psem, accum):
        cid = lax.axis_index("c")
        sid = lax.axis_index("s")
        wid = sid * _NC + cid
        base = wid * per_w
        pltpu.sync_copy(z_h.at[pl.ds(sid * rpt, rpt)],
                        accum.at[pl.ds(sid * rpt, rpt)])
        pltpu.sync_copy(dst_h.at[pl.ds(wid * nchunks, nchunks)], idx_v)
        plsc.subcore_barrier()
        prev = pltpu.async_copy(pay_h.at[pl.ds(base, _CH)],
                                pay_v.at[0], psem)
        for i in range(nchunks):
            nxt = None
            if i + 1 < nchunks:
                nxt = pltpu.async_copy(
                    pay_h.at[pl.ds(base + (i + 1) * _CH, _CH)],
                    pay_v.at[(i + 1) % 2], psem)
            prev.wait()
            pltpu.sync_copy(pay_v.at[i % 2], accum.at[idx_v.at[i]], add=True)
            prev = nxt
        plsc.subcore_barrier()
        pltpu.sync_copy(accum.at[pl.ds(sid * rpt, rpt)],
                        out_h.at[cid, pl.ds(sid * rpt, rpt)])

    f = pl.kernel(
        body,
        out_type=jax.ShapeDtypeStruct((_NC, nacc, 32), jnp.float32),
        mesh=mesh,
        scratch_types=[
            pltpu.VMEM((nchunks, _CH), jnp.int32),
            pltpu.VMEM((2, _CH, 32), jnp.float32),
            pltpu.SemaphoreType.DMA,
            pltpu.VMEM_SHARED((nacc, 32), jnp.float32),
        ],
        compiler_params=pltpu.CompilerParams(use_tc_tiling_on_sc=False),
        interpret=_INTERPRET,
    )
    return f(payload, dst2, zinit)


def _sigmoid(v):
    return 1.0 / (1.0 + jnp.exp(-v))


def _tc_init(x, pos, W1, b1r, tn):
    n = x.shape[0]
    din = x.shape[1]
    grid = (n // tn,)

    def body(x_r, p_r, w_r, b_r, tab_r, posd_r):
        h = jnp.dot(x_r[...], w_r[...]) + b_r[...]
        p = p_r[...]
        z13 = jnp.zeros((tn, 13), jnp.float32)
        tab_r[...] = jnp.concatenate([h, p, z13], axis=1)
        posd_r[...] = jnp.concatenate([p, z13], axis=1)

    return pl.pallas_call(
        body,
        grid=grid,
        in_specs=[
            pl.BlockSpec((tn, din), lambda i: (i, 0)),
            pl.BlockSpec((tn, 3), lambda i: (i, 0)),
            pl.BlockSpec(W1.shape, lambda i: (0, 0)),
            pl.BlockSpec(b1r.shape, lambda i: (0, 0)),
        ],
        out_specs=[
            pl.BlockSpec((tn, 32), lambda i: (i, 0)),
            pl.BlockSpec((tn, 16), lambda i: (i, 0)),
        ],
        out_shape=[
            jax.ShapeDtypeStruct((n, 32), jnp.float32),
            jax.ShapeDtypeStruct((n, 16), jnp.float32),
        ],
        interpret=_INTERPRET,
    )(x, pos, W1, b1r)


def _tc_edgepre(eap, nW1a, nb1r, teb):
    epad = eap.shape[0]
    dedge = eap.shape[1]
    grid = (epad // teb,)

    def body(ea_r, w1a_r, b1_r, out_r):
        out_r[...] = jnp.dot(ea_r[...], w1a_r[...]) + b1_r[...]

    return pl.pallas_call(
        body,
        grid=grid,
        in_specs=[
            pl.BlockSpec((teb, dedge), lambda i: (i, 0)),
            pl.BlockSpec(nW1a.shape, lambda i: (0, 0)),
            pl.BlockSpec(nb1r.shape, lambda i: (0, 0)),
        ],
        out_specs=pl.BlockSpec((teb, 32), lambda i: (i, 0)),
        out_shape=jax.ShapeDtypeStruct((epad, 32), jnp.float32),
        interpret=_INTERPRET,
    )(eap, nW1a, nb1r)


def _tc_edge(gsrc, gdst, ea1, nW1b, nW2, nb2r, tile_m, sel_m, wc_l, teb):
    epad = gsrc.shape[0]
    grid = (epad // teb,)

    def body(gs_r, gd_r, ea_r, w1b_r, w2_r, b2_r, tl_r, sl_r,
             wc_r, out_r):
        gs = gs_r[...]
        hs = gs[:, 0:16]
        rel = gs[:, 16:19] - gd_r[...][:, 0:3]
        dist = jnp.sqrt(jnp.sum(rel * rel, axis=1, keepdims=True) + 1e-12)
        zpre = ea_r[...] + dist * w1b_r[...]
        z = zpre * _sigmoid(zpre)
        wef = jnp.dot(z, w2_r[...]) + b2_r[...]
        hst = jnp.dot(hs, tl_r[...])
        msg = jnp.dot(wef * hst, sl_r[...])
        cw = jnp.dot(msg, wc_r[...])
        wmsg = rel * cw
        ones = jnp.ones((teb, 1), jnp.float32)
        pad = jnp.zeros((teb, 12), jnp.float32)
        out_r[...] = jnp.concatenate([msg, wmsg, ones, pad], axis=1)

    return pl.pallas_call(
        body,
        grid=grid,
        in_specs=[
            pl.BlockSpec((teb, 32), lambda i: (i, 0)),
            pl.BlockSpec((teb, 16), lambda i: (i, 0)),
            pl.BlockSpec((teb, 32), lambda i: (i, 0)),
            pl.BlockSpec(nW1b.shape, lambda i: (0, 0)),
            pl.BlockSpec(nW2.shape, lambda i: (0, 0)),
            pl.BlockSpec(nb2r.shape, lambda i: (0, 0)),
            pl.BlockSpec(tile_m.shape, lambda i: (0, 0)),
            pl.BlockSpec(sel_m.shape, lambda i: (0, 0)),
            pl.BlockSpec(wc_l.shape, lambda i: (0, 0)),
        ],
        out_specs=pl.BlockSpec((teb, 32), lambda i: (i, 0)),
        out_shape=jax.ShapeDtypeStruct((epad, 32), jnp.float32),
        compiler_params=pltpu.CompilerParams(
            dimension_semantics=("arbitrary",)),
        interpret=_INTERPRET,
    )(gsrc, gdst, ea1, nW1b, nW2, nb2r, tile_m, sel_m, wc_l)


def _tc_update(table, agg2, Wr_l, br_l, tn):
    n = table.shape[0]
    nacc = agg2.shape[1]
    grid = (n // tn,)

    def body(tab_r, agg_r, wr_r, br_r, tabo_r, posdo_r):
        agg = agg_r[0] + agg_r[1]
        cnt = agg[:, 19:20]
        deg = jnp.maximum(cnt, 1.0)
        aggh = agg[:, 0:16] / deg
        aggp = agg[:, 16:19] / deg
        tab = tab_r[...]
        h = tab[:, 0:16]
        p = tab[:, 16:19]
        hn = h + jnp.dot(h, wr_r[...]) + aggh + br_r[...]
        pn = p + aggp
        z13 = jnp.zeros((tn, 13), jnp.float32)
        tabo_r[...] = jnp.concatenate([hn, pn, z13], axis=1)
        posdo_r[...] = jnp.concatenate([pn, z13], axis=1)

    return pl.pallas_call(
        body,
        grid=grid,
        in_specs=[
            pl.BlockSpec((tn, 32), lambda i: (i, 0)),
            pl.BlockSpec((2, tn, 32), lambda i: (0, i, 0)),
            pl.BlockSpec(Wr_l.shape, lambda i: (0, 0)),
            pl.BlockSpec(br_l.shape, lambda i: (0, 0)),
        ],
        out_specs=[
            pl.BlockSpec((tn, 32), lambda i: (i, 0)),
            pl.BlockSpec((tn, 16), lambda i: (i, 0)),
        ],
        out_shape=[
            jax.ShapeDtypeStruct((n, 32), jnp.float32),
            jax.ShapeDtypeStruct((n, 16), jnp.float32),
        ],
        interpret=_INTERPRET,
    )(table, agg2, Wr_l, br_l)


def _tc_set2set(table, batch2, W_ih, W_hh, b_lstm_r, Wo1, bo1r, Wo2, bo2r,
                bgraph, msteps):
    n = table.shape[0]
    hdim = 16

    def body(tab_r, bat_r, wih_r, whh_r, bl_r, wo1_r, bo1_r, wo2_r, bo2_r,
             out_r):
        h = tab_r[...][:, 0:hdim]
        bat = bat_r[...]
        ids = lax.broadcasted_iota(jnp.int32, (1, bgraph), 1)
        pm = bat == ids
        qstar = jnp.zeros((bgraph, 2 * hdim), jnp.float32)
        hs = jnp.zeros((bgraph, hdim), jnp.float32)
        cs = jnp.zeros((bgraph, hdim), jnp.float32)
        for _ in range(msteps):
            gates = (jnp.dot(qstar, wih_r[...]) + jnp.dot(hs, whh_r[...])
                     + bl_r[...])
            gi = gates[:, 0:hdim]
            gf = gates[:, hdim:2 * hdim]
            gg = gates[:, 2 * hdim:3 * hdim]
            go = gates[:, 3 * hdim:4 * hdim]
            cs = _sigmoid(gf) * cs + _sigmoid(gi) * jnp.tanh(gg)
            hs = _sigmoid(go) * jnp.tanh(cs)
            s = lax.dot_general(h, hs, (((1,), (1,)), ((), ())))
            masked = jnp.where(pm, s, -jnp.inf)
            emax = jnp.max(masked, axis=0, keepdims=True)
            emax = jnp.where(emax > -jnp.inf, emax, 0.0)
            a = jnp.exp(masked - emax)
            asum = jnp.sum(a, axis=0, keepdims=True)
            asum = jnp.where(asum > 0.0, asum, 1.0)
            an = a / asum
            r = lax.dot_general(an, h, (((0,), (0,)), ((), ())))
            qstar = jnp.concatenate([hs, r], axis=1)
        t = jnp.dot(qstar, wo1_r[...]) + bo1_r[...]
        sil = t * _sigmoid(t)
        out_r[...] = jnp.dot(sil, wo2_r[...]) + bo2_r[...]

    return pl.pallas_call(
        body,
        out_shape=jax.ShapeDtypeStruct((bgraph, 1), jnp.float32),
        interpret=_INTERPRET,
    )(table, batch2, W_ih, W_hh, b_lstm_r, Wo1, bo1r, Wo2, bo2r)


def kernel(x, edge_index, edge_attr, pos, batch, W1, b1, nW1, nb1, nW2, nb2,
           Wr, br, Wc, W_ih, W_hh, b_lstm, Wo1, bo1, Wo2, bo2):
    n, din = x.shape
    e = edge_index.shape[1]
    dedge = edge_attr.shape[1]
    hdim = W1.shape[1]
    nlayers = Wr.shape[0]
    bgraph = 64
    msteps = 3
    tn = 2000

    epad = _cdiv(e, _NW * _CH) * (_NW * _CH)
    nacc = n + 16

    src = edge_index[0].astype(jnp.int32)
    dst = edge_index[1].astype(jnp.int32)
    srcp = jnp.concatenate([src, jnp.zeros((epad - e,), jnp.int32)])
    dstp = jnp.concatenate([dst, jnp.full((epad - e,), n, jnp.int32)])
    eap = jnp.concatenate(
        [edge_attr, jnp.zeros((epad - e, dedge), jnp.float32)], axis=0)
    zinit = jnp.zeros((nacc, 32), jnp.float32)

    eye = jnp.eye(hdim, dtype=jnp.float32)
    tile_m = jnp.tile(eye, (1, hdim))
    sel_m = jnp.repeat(eye, hdim, axis=0)

    nW1a = nW1[:dedge]
    nW1b = nW1[dedge:dedge + 1]
    nb1r = nb1.reshape(1, -1)
    nb2r = nb2.reshape(1, -1)

    table, posd = _tc_init(x, pos, W1, b1.reshape(1, -1), tn)
    ea1 = _tc_edgepre(eap, nW1a, nb1r, 2048)
    for l in range(nlayers):
        gsrc, gdst = _sc_gather(table, posd, srcp, dstp)
        payload = _tc_edge(gsrc, gdst, ea1, nW1b, nW2, nb2r,
                           tile_m, sel_m, Wc[l], 2048)
        agg2 = _sc_scatter(payload, dstp, zinit)
        table, posd = _tc_update(table, agg2, Wr[l], br[l].reshape(1, -1), tn)

    out = _tc_set2set(table, batch.reshape(-1, 1).astype(jnp.int32),
                      W_ih, W_hh, b_lstm.reshape(1, -1), Wo1,
                      bo1.reshape(1, -1), Wo2, bo2.reshape(1, 1),
                      bgraph, msteps)
    return out.reshape(-1)


# speedup vs baseline: 3.0541x; 1.0476x over previous
import functools

import jax
import jax.numpy as jnp
from jax import lax
from jax.experimental import pallas as pl
from jax.experimental.pallas import tpu as pltpu
from jax.experimental.pallas import tpu_sc as plsc

_NC = 2
_NS = 16
_NW = _NC * _NS
_CH = 128

_INTERPRET = False


def _cdiv(a, b):
    return (a + b - 1) // b


_GDEPTH = 8


def _sc_gather(table, posd, srcp, dstp):
    epad = srcp.shape[0]
    per_w = epad // _NW
    nchunks = per_w // _CH
    ngroups = nchunks // _GDEPTH
    src2 = srcp.reshape(-1, _CH)
    dst2 = dstp.reshape(-1, _CH)
    mesh = plsc.VectorSubcoreMesh(core_axis_name="c", subcore_axis_name="s")

    def body(table_h, posd_h, src_h, dst_h, gsrc_h, gdst_h,
             idx_s, idx_d, bufs, bufd, gsem, wsem):
        wid = lax.axis_index("s") * _NC + lax.axis_index("c")
        base = wid * per_w
        crow = wid * nchunks
        pltpu.sync_copy(src_h.at[pl.ds(crow, nchunks)], idx_s)
        pltpu.sync_copy(dst_h.at[pl.ds(crow, nchunks)], idx_d)
        wdescs = []
        for g in range(ngroups):
            for d in wdescs:
                d.wait()
            wdescs = []
            gdescs = []
            for b in range(_GDEPTH):
                j = g * _GDEPTH + b
                gdescs.append(pltpu.async_copy(
                    table_h.at[idx_s.at[j]], bufs.at[b], gsem))
                gdescs.append(pltpu.async_copy(
                    posd_h.at[idx_d.at[j]], bufd.at[b], gsem))
            for d in gdescs:
                d.wait()
            for b in range(_GDEPTH):
                off = base + (g * _GDEPTH + b) * _CH
                wdescs.append(pltpu.async_copy(
                    bufs.at[b], gsrc_h.at[pl.ds(off, _CH)], wsem))
                wdescs.append(pltpu.async_copy(
                    bufd.at[b], gdst_h.at[pl.ds(off, _CH)], wsem))
        for d in wdescs:
            d.wait()

    f = pl.kernel(
        body,
        out_type=(jax.ShapeDtypeStruct((epad, 32), jnp.float32),
                  jax.ShapeDtypeStruct((epad, 16), jnp.float32)),
        mesh=mesh,
        scratch_types=[
            pltpu.VMEM((nchunks, _CH), jnp.int32),
            pltpu.VMEM((nchunks, _CH), jnp.int32),
            pltpu.VMEM((_GDEPTH, _CH, 32), jnp.float32),
            pltpu.VMEM((_GDEPTH, _CH, 16), jnp.float32),
            pltpu.SemaphoreType.DMA,
            pltpu.SemaphoreType.DMA,
        ],
        compiler_params=pltpu.CompilerParams(use_tc_tiling_on_sc=False),
        interpret=_INTERPRET,
    )
    return f(table, posd, src2, dst2)


def _sc_scatter(payload, dstp, zinit):
    epad = payload.shape[0]
    nacc = zinit.shape[0]
    per_w = epad // _NW
    nchunks = per_w // _CH
    rpt = nacc // _NS
    mesh = plsc.VectorSubcoreMesh(core_axis_name="c", subcore_axis_name="s")

    dst2 = dstp.reshape(-1, _CH)

    def body(pay_h, dst_h, z_h, out_h, idx_v, pay_v, psem, accum):
        cid = lax.axis_index("c")
        sid = lax.axis_index("s")
        wid = sid * _NC + cid
        base = wid * per_w
        pltpu.sync_copy(z_h.at[pl.ds(sid * rpt, rpt)],
                        accum.at[pl.ds(sid * rpt, rpt)])
        pltpu.sync_copy(dst_h.at[pl.ds(wid * nchunks, nchunks)], idx_v)
        plsc.subcore_barrier()
        prev = pltpu.async_copy(pay_h.at[pl.ds(base, _CH)],
                                pay_v.at[0], psem)
        for i in range(nchunks):
            nxt = None
            if i + 1 < nchunks:
                nxt = pltpu.async_copy(
                    pay_h.at[pl.ds(base + (i + 1) * _CH, _CH)],
                    pay_v.at[(i + 1) % 2], psem)
            prev.wait()
            pltpu.sync_copy(pay_v.at[i % 2], accum.at[idx_v.at[i]], add=True)
            prev = nxt
        plsc.subcore_barrier()
        pltpu.sync_copy(accum.at[pl.ds(sid * rpt, rpt)],
                        out_h.at[cid, pl.ds(sid * rpt, rpt)])

    f = pl.kernel(
        body,
        out_type=jax.ShapeDtypeStruct((_NC, nacc, 32), jnp.float32),
        mesh=mesh,
        scratch_types=[
            pltpu.VMEM((nchunks, _CH), jnp.int32),
            pltpu.VMEM((2, _CH, 32), jnp.float32),
            pltpu.SemaphoreType.DMA,
            pltpu.VMEM_SHARED((nacc, 32), jnp.float32),
        ],
        compiler_params=pltpu.CompilerParams(use_tc_tiling_on_sc=False),
        interpret=_INTERPRET,
    )
    return f(payload, dst2, zinit)


def _sigmoid(v):
    return 1.0 / (1.0 + jnp.exp(-v))


def _tc_init(x, pos, W1, b1r, tn):
    n = x.shape[0]
    din = x.shape[1]
    grid = (n // tn,)

    def body(x_r, p_r, w_r, b_r, tab_r, posd_r):
        h = jnp.dot(x_r[...], w_r[...]) + b_r[...]
        p = p_r[...]
        z13 = jnp.zeros((tn, 13), jnp.float32)
        tab_r[...] = jnp.concatenate([h, p, z13], axis=1)
        posd_r[...] = jnp.concatenate([p, z13], axis=1)

    return pl.pallas_call(
        body,
        grid=grid,
        in_specs=[
            pl.BlockSpec((tn, din), lambda i: (i, 0)),
            pl.BlockSpec((tn, 3), lambda i: (i, 0)),
            pl.BlockSpec(W1.shape, lambda i: (0, 0)),
            pl.BlockSpec(b1r.shape, lambda i: (0, 0)),
        ],
        out_specs=[
            pl.BlockSpec((tn, 32), lambda i: (i, 0)),
            pl.BlockSpec((tn, 16), lambda i: (i, 0)),
        ],
        out_shape=[
            jax.ShapeDtypeStruct((n, 32), jnp.float32),
            jax.ShapeDtypeStruct((n, 16), jnp.float32),
        ],
        interpret=_INTERPRET,
    )(x, pos, W1, b1r)


def _tc_edgepre(eap, nW1a, nb1r, teb):
    epad = eap.shape[0]
    dedge = eap.shape[1]
    grid = (epad // teb,)

    def body(ea_r, w1a_r, b1_r, out_r):
        out_r[...] = jnp.dot(ea_r[...], w1a_r[...]) + b1_r[...]

    return pl.pallas_call(
        body,
        grid=grid,
        in_specs=[
            pl.BlockSpec((teb, dedge), lambda i: (i, 0)),
            pl.BlockSpec(nW1a.shape, lambda i: (0, 0)),
            pl.BlockSpec(nb1r.shape, lambda i: (0, 0)),
        ],
        out_specs=pl.BlockSpec((teb, 32), lambda i: (i, 0)),
        out_shape=jax.ShapeDtypeStruct((epad, 32), jnp.float32),
        interpret=_INTERPRET,
    )(eap, nW1a, nb1r)


def _tc_edge(gsrc, gdst, ea1, nW1b, nW2, nb2d, tile_m, sel_m, wc_l, teb):
    epad = gsrc.shape[0]
    grid = (epad // teb,)

    def body(gs_r, gd_r, ea_r, w1b_r, w2_r, b2d_r, tl_r, sl_r,
             wc_r, out_r):
        gs = gs_r[...]
        hs = gs[:, 0:16]
        rel = gs[:, 16:19] - gd_r[...][:, 0:3]
        dist = jnp.sqrt(jnp.sum(rel * rel, axis=1, keepdims=True) + 1e-12)
        zpre = ea_r[...] + dist * w1b_r[...]
        z = zpre * _sigmoid(zpre)
        wef = jnp.dot(z, w2_r[...])
        hst = jnp.dot(hs, tl_r[...])
        msg = jnp.dot(wef * hst, sl_r[...]) + jnp.dot(hs, b2d_r[...])
        cw = jnp.dot(msg, wc_r[...])
        wmsg = rel * cw
        ones = jnp.ones((teb, 1), jnp.float32)
        pad = jnp.zeros((teb, 12), jnp.float32)
        out_r[...] = jnp.concatenate([msg, wmsg, ones, pad], axis=1)

    return pl.pallas_call(
        body,
        grid=grid,
        in_specs=[
            pl.BlockSpec((teb, 32), lambda i: (i, 0)),
            pl.BlockSpec((teb, 16), lambda i: (i, 0)),
            pl.BlockSpec((teb, 32), lambda i: (i, 0)),
            pl.BlockSpec(nW1b.shape, lambda i: (0, 0)),
            pl.BlockSpec(nW2.shape, lambda i: (0, 0)),
            pl.BlockSpec(nb2d.shape, lambda i: (0, 0)),
            pl.BlockSpec(tile_m.shape, lambda i: (0, 0)),
            pl.BlockSpec(sel_m.shape, lambda i: (0, 0)),
            pl.BlockSpec(wc_l.shape, lambda i: (0, 0)),
        ],
        out_specs=pl.BlockSpec((teb, 32), lambda i: (i, 0)),
        out_shape=jax.ShapeDtypeStruct((epad, 32), jnp.float32),
        compiler_params=pltpu.CompilerParams(
            dimension_semantics=("parallel",)),
        interpret=_INTERPRET,
    )(gsrc, gdst, ea1, nW1b, nW2, nb2d, tile_m, sel_m, wc_l)


def _tc_update(table, agg2, Wr_l, br_l, tn):
    n = table.shape[0]
    nacc = agg2.shape[1]
    grid = (n // tn,)

    def body(tab_r, agg_r, wr_r, br_r, tabo_r, posdo_r):
        agg = agg_r[0] + agg_r[1]
        cnt = agg[:, 19:20]
        deg = jnp.maximum(cnt, 1.0)
        aggh = agg[:, 0:16] / deg
        aggp = agg[:, 16:19] / deg
        tab = tab_r[...]
        h = tab[:, 0:16]
        p = tab[:, 16:19]
        hn = h + jnp.dot(h, wr_r[...]) + aggh + br_r[...]
        pn = p + aggp
        z13 = jnp.zeros((tn, 13), jnp.float32)
        tabo_r[...] = jnp.concatenate([hn, pn, z13], axis=1)
        posdo_r[...] = jnp.concatenate([pn, z13], axis=1)

    return pl.pallas_call(
        body,
        grid=grid,
        in_specs=[
            pl.BlockSpec((tn, 32), lambda i: (i, 0)),
            pl.BlockSpec((2, tn, 32), lambda i: (0, i, 0)),
            pl.BlockSpec(Wr_l.shape, lambda i: (0, 0)),
            pl.BlockSpec(br_l.shape, lambda i: (0, 0)),
        ],
        out_specs=[
            pl.BlockSpec((tn, 32), lambda i: (i, 0)),
            pl.BlockSpec((tn, 16), lambda i: (i, 0)),
        ],
        out_shape=[
            jax.ShapeDtypeStruct((n, 32), jnp.float32),
            jax.ShapeDtypeStruct((n, 16), jnp.float32),
        ],
        interpret=_INTERPRET,
    )(table, agg2, Wr_l, br_l)


def _tc_set2set(table, batch2, W_ih, W_hh, b_lstm_r, Wo1, bo1r, Wo2, bo2r,
                bgraph, msteps):
    n = table.shape[0]
    hdim = 16

    def body(tab_r, bat_r, wih_r, whh_r, bl_r, wo1_r, bo1_r, wo2_r, bo2_r,
             out_r):
        h = tab_r[...][:, 0:hdim]
        bat = bat_r[...]
        ids = lax.broadcasted_iota(jnp.int32, (1, bgraph), 1)
        pm = bat == ids
        qstar = jnp.zeros((bgraph, 2 * hdim), jnp.float32)
        hs = jnp.zeros((bgraph, hdim), jnp.float32)
        cs = jnp.zeros((bgraph, hdim), jnp.float32)
        for _ in range(msteps):
            gates = (jnp.dot(qstar, wih_r[...]) + jnp.dot(hs, whh_r[...])
                     + bl_r[...])
            gi = gates[:, 0:hdim]
            gf = gates[:, hdim:2 * hdim]
            gg = gates[:, 2 * hdim:3 * hdim]
            go = gates[:, 3 * hdim:4 * hdim]
            cs = _sigmoid(gf) * cs + _sigmoid(gi) * jnp.tanh(gg)
            hs = _sigmoid(go) * jnp.tanh(cs)
            s = lax.dot_general(h, hs, (((1,), (1,)), ((), ())))
            masked = jnp.where(pm, s, -jnp.inf)
            emax = jnp.max(masked, axis=0, keepdims=True)
            emax = jnp.where(emax > -jnp.inf, emax, 0.0)
            a = jnp.exp(masked - emax)
            asum = jnp.sum(a, axis=0, keepdims=True)
            asum = jnp.where(asum > 0.0, asum, 1.0)
            an = a / asum
            r = lax.dot_general(an, h, (((0,), (0,)), ((), ())))
            qstar = jnp.concatenate([hs, r], axis=1)
        t = jnp.dot(qstar, wo1_r[...]) + bo1_r[...]
        sil = t * _sigmoid(t)
        out_r[...] = jnp.dot(sil, wo2_r[...]) + bo2_r[...]

    return pl.pallas_call(
        body,
        out_shape=jax.ShapeDtypeStruct((bgraph, 1), jnp.float32),
        interpret=_INTERPRET,
    )(table, batch2, W_ih, W_hh, b_lstm_r, Wo1, bo1r, Wo2, bo2r)


def kernel(x, edge_index, edge_attr, pos, batch, W1, b1, nW1, nb1, nW2, nb2,
           Wr, br, Wc, W_ih, W_hh, b_lstm, Wo1, bo1, Wo2, bo2):
    n, din = x.shape
    e = edge_index.shape[1]
    dedge = edge_attr.shape[1]
    hdim = W1.shape[1]
    nlayers = Wr.shape[0]
    bgraph = 64
    msteps = 3
    tn = 2000

    epad = _cdiv(e, _NW * _CH) * (_NW * _CH)
    nacc = n + 16

    src = edge_index[0].astype(jnp.int32)
    dst = edge_index[1].astype(jnp.int32)
    srcp = jnp.concatenate([src, jnp.zeros((epad - e,), jnp.int32)])
    dstp = jnp.concatenate([dst, jnp.full((epad - e,), n, jnp.int32)])
    eap = jnp.concatenate(
        [edge_attr, jnp.zeros((epad - e, dedge), jnp.float32)], axis=0)
    zinit = jnp.zeros((nacc, 32), jnp.float32)

    eye = jnp.eye(hdim, dtype=jnp.float32)
    tile_m = jnp.tile(eye, (1, hdim))
    sel_m = jnp.repeat(eye, hdim, axis=0)

    nW1a = nW1[:dedge]
    nW1b = nW1[dedge:dedge + 1]
    nb1r = nb1.reshape(1, -1)
    nb2d = nb2.reshape(hdim, hdim).T

    table, posd = _tc_init(x, pos, W1, b1.reshape(1, -1), tn)
    ea1 = _tc_edgepre(eap, nW1a, nb1r, 2048)
    for l in range(nlayers):
        gsrc, gdst = _sc_gather(table, posd, srcp, dstp)
        payload = _tc_edge(gsrc, gdst, ea1, nW1b, nW2, nb2d,
                           tile_m, sel_m, Wc[l], 4096)
        agg2 = _sc_scatter(payload, dstp, zinit)
        table, posd = _tc_update(table, agg2, Wr[l], br[l].reshape(1, -1), tn)

    out = _tc_set2set(table, batch.reshape(-1, 1).astype(jnp.int32),
                      W_ih, W_hh, b_lstm.reshape(1, -1), Wo1,
                      bo1.reshape(1, -1), Wo2, bo2.reshape(1, 1),
                      bgraph, msteps)
    return out.reshape(-1)
